# baseline (speedup 1.0000x reference)
import functools

import jax
import jax.numpy as jnp
from jax.experimental import pallas as pl
from jax.experimental.pallas import tpu as pltpu

_SPARSITY = 0.01
_BISECT_ITERS_PER_STEP = 2


def _bisect_iters(load, lo, hi, kf, n, f_tiles):
    for _ in range(n):
        mid16 = (0.5 * (lo + hi)).astype(jnp.bfloat16)
        cnt = jnp.zeros_like(lo)
        for j in range(f_tiles):
            msk = jnp.where(load(j) >= mid16,
                            jnp.bfloat16(1), jnp.bfloat16(0))
            cnt = cnt + jnp.sum(msk, axis=1,
                                keepdims=True).astype(jnp.float32)
        pred = cnt >= kf
        mid = mid16.astype(jnp.float32)
        lo = jnp.where(pred, mid, lo)
        hi = jnp.where(pred, hi, mid)
    return lo, hi


def _k1_body(hs_ref, fd_ref, m_ref, ge_ref, agg_ref, sacc_ref,
             acts_ref, max_ref, lo_ref, hi_ref,
             *, nt, f_tiles, rpt, f_tile, topk, n_groups, tpb):
    t = pl.program_id(0)
    f = pl.program_id(1)
    kf = jnp.float32(topk)

    @pl.when(t < nt)
    def _matmul():
        a = jax.lax.dot_general(hs_ref[...], fd_ref[...],
                                (((1,), (1,)), ((), ())),
                                preferred_element_type=jnp.float32)
        a = jnp.maximum(a, 0.0)
        buf = jax.lax.rem(t, 2)
        acts_ref[buf, f] = a.astype(jnp.bfloat16)
        max_ref[buf, f] = jnp.max(a, axis=1, keepdims=True)

    @pl.when(t > 0)
    def _select():
        p = t - 1
        buf = jax.lax.rem(p, 2)
        load = lambda j: acts_ref[buf, j]

        @pl.when(f == 0)
        def _init():
            hi = max_ref[buf, 0]
            for j in range(1, f_tiles):
                hi = jnp.maximum(hi, max_ref[buf, j])
            lo, hi = _bisect_iters(load, jnp.zeros_like(hi), hi, kf,
                                   _BISECT_ITERS_PER_STEP, f_tiles)
            lo_ref[...], hi_ref[...] = lo, hi

        @pl.when(jnp.logical_and(f > 0, f < f_tiles - 1))
        def _iter():
            lo, hi = _bisect_iters(load, lo_ref[...], hi_ref[...], kf,
                                   _BISECT_ITERS_PER_STEP, f_tiles)
            lo_ref[...], hi_ref[...] = lo, hi

        @pl.when(f == f_tiles - 1)
        def _finish():
            lo, _ = _bisect_iters(load, lo_ref[...], hi_ref[...], kf,
                                  _BISECT_ITERS_PER_STEP, f_tiles)
            lo16 = lo.astype(jnp.bfloat16)
            agg = jnp.zeros(agg_ref.shape, jnp.float32)
            ssum = jnp.zeros((agg_ref.shape[0], 1), jnp.float32)
            for j in range(rpt):
                va = load(j)
                vb = load(rpt + j)
                vc = load(2 * rpt + j)
                va = jnp.where(va >= lo16, va, jnp.zeros_like(va))
                vb = jnp.where(vb >= lo16, vb, jnp.zeros_like(vb))
                vc = jnp.where(vc >= lo16, vc, jnp.zeros_like(vc))
                med = jnp.maximum(jnp.minimum(va, vb),
                                  jnp.minimum(jnp.maximum(va, vb), vc))
                agg = agg + jax.lax.dot_general(
                    med, ge_ref[j * f_tile:(j + 1) * f_tile, :],
                    (((1,), (0,)), ((), ())),
                    preferred_element_type=jnp.float32)
                sig_d = jax.nn.sigmoid(med) - jnp.bfloat16(0.5)
                ssum = ssum + jnp.sum(sig_d, axis=1,
                                      keepdims=True).astype(jnp.float32)
            agg_ref[...] = agg

            m = m_ref[...]
            stok = ((ssum + 0.5 * n_groups) * (1.0 / n_groups)) * m
            num = jnp.sum(stok)
            den = jnp.sum(m)
            lane = jax.lax.broadcasted_iota(jnp.int32, (1, 1, 128), 2)
            vec = jnp.where(lane == 0, num, jnp.where(lane == 1, den, 0.0))
            prev = jnp.where(jax.lax.rem(p, tpb) == 0,
                             jnp.zeros((1, 1, 128), jnp.float32),
                             sacc_ref[...])
            sacc_ref[...] = prev + vec


def _k2_body(hs_ref, agg_ref, w_ref, gam_ref, bet_ref, g_ref, out_ref):
    fc = jax.lax.dot_general(agg_ref[...], w_ref[...],
                             (((1,), (1,)), ((), ())),
                             preferred_element_type=jnp.float32)
    mod = hs_ref[...] + g_ref[...] * fc
    mu = jnp.mean(mod, axis=1, keepdims=True)
    d = mod - mu
    var = jnp.mean(d * d, axis=1, keepdims=True)
    out_ref[...] = d * jax.lax.rsqrt(var + 1e-5) * gam_ref[...] + bet_ref[...]


def kernel(hidden_states, attention_mask, feature_directions, group_embeddings,
           mixer_W, ln_gamma, ln_beta, safety_threshold):
    B, S, D = hidden_states.shape
    N = feature_directions.shape[0]
    G, F = group_embeddings.shape
    R = N // G
    topk = max(1, int(N * _SPARSITY))

    TB = 512
    while (S % TB) != 0:
        TB //= 2
    NT = (B * S) // TB
    TPB = NT // B
    F_TILE = min(2048, G)
    while (G % F_TILE) != 0:
        F_TILE //= 2
    RPT = G // F_TILE
    F_TILES = R * RPT

    hs2 = hidden_states.reshape(B * S, D)
    mask2 = attention_mask.reshape(B * S, 1)
    hs_bf = (hs2 * mask2).astype(jnp.bfloat16)
    fd_perm = (feature_directions.reshape(G, R, D).transpose(1, 0, 2)
               .reshape(N, D).astype(jnp.bfloat16))

    last_t = NT - 1

    k1 = pl.pallas_call(
        functools.partial(_k1_body, nt=NT, f_tiles=F_TILES, rpt=RPT,
                          f_tile=F_TILE, topk=topk, n_groups=G, tpb=TPB),
        grid=(NT + 1, F_TILES),
        in_specs=[
            pl.BlockSpec((TB, D), lambda t, f: (jnp.minimum(t, last_t), 0)),
            pl.BlockSpec((F_TILE, D), lambda t, f: (f, 0)),
            pl.BlockSpec((TB, 1), lambda t, f: (jnp.maximum(t - 1, 0), 0)),
            pl.BlockSpec((G, F), lambda t, f: (0, 0)),
        ],
        out_specs=[
            pl.BlockSpec((TB, F), lambda t, f: (jnp.maximum(t - 1, 0), 0)),
            pl.BlockSpec((1, 1, 128),
                         lambda t, f: (jnp.maximum(t - 1, 0) // TPB, 0, 0)),
        ],
        out_shape=[
            jax.ShapeDtypeStruct((B * S, F), jnp.float32),
            jax.ShapeDtypeStruct((B, 1, 128), jnp.float32),
        ],
        scratch_shapes=[
            pltpu.VMEM((2, F_TILES, TB, F_TILE), jnp.bfloat16),
            pltpu.VMEM((2, F_TILES, TB, 1), jnp.float32),
            pltpu.VMEM((TB, 1), jnp.float32),
            pltpu.VMEM((TB, 1), jnp.float32),
        ],
        compiler_params=pltpu.CompilerParams(
            dimension_semantics=("arbitrary", "arbitrary")),
    )
    agg, sacc = k1(hs_bf, fd_perm, mask2,
                   group_embeddings.astype(jnp.bfloat16))

    num = sacc[:, 0, 0]
    den = jnp.maximum(sacc[:, 0, 1], 1.0)
    safety_scores = num / den
    gates = jax.nn.sigmoid(safety_scores - safety_threshold)
    g_tok = jnp.broadcast_to(gates[:, None], (B, S)).reshape(B * S, 1)

    TB2 = 512
    while ((B * S) % TB2) != 0:
        TB2 //= 2

    k2 = pl.pallas_call(
        _k2_body,
        grid=((B * S) // TB2,),
        in_specs=[
            pl.BlockSpec((TB2, D), lambda t: (t, 0)),
            pl.BlockSpec((TB2, F), lambda t: (t, 0)),
            pl.BlockSpec((D, F), lambda t: (0, 0)),
            pl.BlockSpec((1, D), lambda t: (0, 0)),
            pl.BlockSpec((1, D), lambda t: (0, 0)),
            pl.BlockSpec((TB2, 1), lambda t: (t, 0)),
        ],
        out_specs=pl.BlockSpec((TB2, D), lambda t: (t, 0)),
        out_shape=jax.ShapeDtypeStruct((B * S, D), jnp.float32),
        compiler_params=pltpu.CompilerParams(
            dimension_semantics=("arbitrary",)),
    )
    out2 = k2(hs2, agg, mixer_W, ln_gamma.reshape(1, D), ln_beta.reshape(1, D),
              g_tok)

    return out2.reshape(B, S, D), safety_scores


# speedup vs baseline: 1.2071x; 1.2071x over previous
import functools

import jax
import jax.numpy as jnp
from jax.experimental import pallas as pl
from jax.experimental.pallas import tpu as pltpu

_SPARSITY = 0.01
_BISECT_ITERS = 8


def _k1_body(hs_ref, fd_ref, m_ref, ge_ref, agg_ref, sacc_ref, acts_ref,
             max_ref, *, f_tiles, rpt, f_tile, topk, n_groups, tpb):
    t = pl.program_id(0)
    f = pl.program_id(1)

    a = jax.lax.dot_general(hs_ref[...], fd_ref[...],
                            (((1,), (1,)), ((), ())),
                            preferred_element_type=jnp.float32)
    a = jnp.maximum(a, 0.0)
    acts_ref[f] = a.astype(jnp.bfloat16)
    max_ref[f] = jnp.max(a, axis=1, keepdims=True)

    @pl.when(f == f_tiles - 1)
    def _finish():
        hi = max_ref[0]
        for j in range(1, f_tiles):
            hi = jnp.maximum(hi, max_ref[j])
        lo = jnp.zeros_like(hi)
        kf = jnp.float32(topk)

        def body(_, lh):
            lo, hi = lh
            mid16 = (0.5 * (lo + hi)).astype(jnp.bfloat16)
            cnt = jnp.zeros_like(lo)
            for j in range(f_tiles):
                msk = jnp.where(acts_ref[j] >= mid16,
                                jnp.bfloat16(1), jnp.bfloat16(0))
                cnt = cnt + jnp.sum(msk, axis=1,
                                    keepdims=True).astype(jnp.float32)
            pred = cnt >= kf
            mid = mid16.astype(jnp.float32)
            return jnp.where(pred, mid, lo), jnp.where(pred, hi, mid)

        lo, hi = jax.lax.fori_loop(0, _BISECT_ITERS, body, (lo, hi))
        lo16 = lo.astype(jnp.bfloat16)

        agg = jnp.zeros(agg_ref.shape, jnp.float32)
        ssum = jnp.zeros((agg_ref.shape[0], 1), jnp.float32)
        for j in range(rpt):
            va = acts_ref[j]
            vb = acts_ref[rpt + j]
            vc = acts_ref[2 * rpt + j]
            va = jnp.where(va >= lo16, va, jnp.zeros_like(va))
            vb = jnp.where(vb >= lo16, vb, jnp.zeros_like(vb))
            vc = jnp.where(vc >= lo16, vc, jnp.zeros_like(vc))
            med = (jnp.maximum(jnp.minimum(va, vb),
                               jnp.minimum(jnp.maximum(va, vb), vc)))
            agg = agg + jax.lax.dot_general(
                med, ge_ref[j * f_tile:(j + 1) * f_tile, :],
                (((1,), (0,)), ((), ())),
                preferred_element_type=jnp.float32)
            sig_d = jax.nn.sigmoid(med) - jnp.bfloat16(0.5)
            ssum = ssum + jnp.sum(sig_d, axis=1,
                                  keepdims=True).astype(jnp.float32)
        agg_ref[...] = agg

        stok = ((ssum + 0.5 * n_groups) * (1.0 / n_groups)) * m_ref[...]
        num = jnp.sum(stok)
        den = jnp.sum(m_ref[...])
        lane = jax.lax.broadcasted_iota(jnp.int32, (1, 1, 128), 2)
        vec = jnp.where(lane == 0, num, jnp.where(lane == 1, den, 0.0))
        prev = jnp.where(t % tpb == 0, jnp.zeros((1, 1, 128), jnp.float32),
                         sacc_ref[...])
        sacc_ref[...] = prev + vec


def _k2_body(hs_ref, agg_ref, w_ref, gam_ref, bet_ref, g_ref, out_ref):
    fc = jax.lax.dot_general(agg_ref[...], w_ref[...],
                             (((1,), (1,)), ((), ())),
                             preferred_element_type=jnp.float32)
    mod = hs_ref[...] + g_ref[...] * fc
    mu = jnp.mean(mod, axis=1, keepdims=True)
    d = mod - mu
    var = jnp.mean(d * d, axis=1, keepdims=True)
    out_ref[...] = d * jax.lax.rsqrt(var + 1e-5) * gam_ref[...] + bet_ref[...]


def kernel(hidden_states, attention_mask, feature_directions, group_embeddings,
           mixer_W, ln_gamma, ln_beta, safety_threshold):
    B, S, D = hidden_states.shape
    N = feature_directions.shape[0]
    G, F = group_embeddings.shape
    R = N // G
    topk = max(1, int(N * _SPARSITY))

    TB = 512
    while (S % TB) != 0:
        TB //= 2
    NT = (B * S) // TB
    TPB = NT // B
    F_TILE = min(4096, G)
    while (G % F_TILE) != 0:
        F_TILE //= 2
    RPT = G // F_TILE
    F_TILES = R * RPT

    hs2 = hidden_states.reshape(B * S, D)
    mask2 = attention_mask.reshape(B * S, 1)
    hs_bf = (hs2 * mask2).astype(jnp.bfloat16)
    fd_perm = (feature_directions.reshape(G, R, D).transpose(1, 0, 2)
               .reshape(N, D).astype(jnp.bfloat16))

    k1 = pl.pallas_call(
        functools.partial(_k1_body, f_tiles=F_TILES, rpt=RPT, f_tile=F_TILE,
                          topk=topk, n_groups=G, tpb=TPB),
        grid=(NT, F_TILES),
        in_specs=[
            pl.BlockSpec((TB, D), lambda t, f: (t, 0)),
            pl.BlockSpec((F_TILE, D), lambda t, f: (f, 0)),
            pl.BlockSpec((TB, 1), lambda t, f: (t, 0)),
            pl.BlockSpec((G, F), lambda t, f: (0, 0)),
        ],
        out_specs=[
            pl.BlockSpec((TB, F), lambda t, f: (t, 0)),
            pl.BlockSpec((1, 1, 128), lambda t, f: (t // TPB, 0, 0)),
        ],
        out_shape=[
            jax.ShapeDtypeStruct((B * S, F), jnp.float32),
            jax.ShapeDtypeStruct((B, 1, 128), jnp.float32),
        ],
        scratch_shapes=[pltpu.VMEM((F_TILES, TB, F_TILE), jnp.bfloat16),
                        pltpu.VMEM((F_TILES, TB, 1), jnp.float32)],
        compiler_params=pltpu.CompilerParams(
            dimension_semantics=("arbitrary", "arbitrary")),
    )
    agg, sacc = k1(hs_bf, fd_perm, mask2,
                   group_embeddings.astype(jnp.bfloat16))

    num = sacc[:, 0, 0]
    den = jnp.maximum(sacc[:, 0, 1], 1.0)
    safety_scores = num / den
    gates = jax.nn.sigmoid(safety_scores - safety_threshold)
    g_tok = jnp.broadcast_to(gates[:, None], (B, S)).reshape(B * S, 1)

    TB2 = 512
    while ((B * S) % TB2) != 0:
        TB2 //= 2

    k2 = pl.pallas_call(
        _k2_body,
        grid=((B * S) // TB2,),
        in_specs=[
            pl.BlockSpec((TB2, D), lambda t: (t, 0)),
            pl.BlockSpec((TB2, F), lambda t: (t, 0)),
            pl.BlockSpec((D, F), lambda t: (0, 0)),
            pl.BlockSpec((1, D), lambda t: (0, 0)),
            pl.BlockSpec((1, D), lambda t: (0, 0)),
            pl.BlockSpec((TB2, 1), lambda t: (t, 0)),
        ],
        out_specs=pl.BlockSpec((TB2, D), lambda t: (t, 0)),
        out_shape=jax.ShapeDtypeStruct((B * S, D), jnp.float32),
        compiler_params=pltpu.CompilerParams(
            dimension_semantics=("arbitrary",)),
    )
    out2 = k2(hs2, agg, mixer_W, ln_gamma.reshape(1, D), ln_beta.reshape(1, D),
              g_tok)

    return out2.reshape(B, S, D), safety_scores


# speedup vs baseline: 1.2197x; 1.0104x over previous
import functools

import jax
import jax.numpy as jnp
from jax.experimental import pallas as pl
from jax.experimental.pallas import tpu as pltpu

_SPARSITY = 0.01
_BISECT_ITERS = 8


def _k1_body(hs_ref, fd_ref, m_ref, ge_ref, agg_ref, sacc_ref, acts_ref,
             max_ref, *, f_tiles, rpt, f_tile, topk, n_groups, tpb):
    t = pl.program_id(0)
    f = pl.program_id(1)

    a = jax.lax.dot_general(hs_ref[...], fd_ref[...],
                            (((1,), (1,)), ((), ())),
                            preferred_element_type=jnp.float32)
    a = jnp.maximum(a, 0.0)
    acts_ref[f] = a.astype(jnp.bfloat16)
    max_ref[f] = jnp.max(a, axis=1, keepdims=True)

    @pl.when(f == f_tiles - 1)
    def _finish():
        hi = max_ref[0]
        for j in range(1, f_tiles):
            hi = jnp.maximum(hi, max_ref[j])
        lo = jnp.zeros_like(hi)
        kf = jnp.float32(topk)

        def body(_, lh):
            lo, hi = lh
            mid16 = (0.5 * (lo + hi)).astype(jnp.bfloat16)
            cnt = jnp.zeros_like(lo)
            for j in range(f_tiles):
                msk = jnp.where(acts_ref[j] >= mid16,
                                jnp.bfloat16(1), jnp.bfloat16(0))
                cnt = cnt + jnp.sum(msk, axis=1,
                                    keepdims=True).astype(jnp.float32)
            pred = cnt >= kf
            mid = mid16.astype(jnp.float32)
            return jnp.where(pred, mid, lo), jnp.where(pred, hi, mid)

        lo, hi = jax.lax.fori_loop(0, _BISECT_ITERS, body, (lo, hi))
        lo16 = lo.astype(jnp.bfloat16)

        agg = jnp.zeros(agg_ref.shape, jnp.float32)
        ssum = jnp.zeros((agg_ref.shape[0], 1), jnp.float32)
        for j in range(rpt):
            va = acts_ref[j]
            vb = acts_ref[rpt + j]
            vc = acts_ref[2 * rpt + j]
            va = jnp.where(va >= lo16, va, jnp.zeros_like(va))
            vb = jnp.where(vb >= lo16, vb, jnp.zeros_like(vb))
            vc = jnp.where(vc >= lo16, vc, jnp.zeros_like(vc))
            med = (jnp.maximum(jnp.minimum(va, vb),
                               jnp.minimum(jnp.maximum(va, vb), vc)))
            agg = agg + jax.lax.dot_general(
                med, ge_ref[j * f_tile:(j + 1) * f_tile, :],
                (((1,), (0,)), ((), ())),
                preferred_element_type=jnp.float32)
            sig_d = jax.nn.sigmoid(med) - jnp.bfloat16(0.5)
            ssum = ssum + jnp.sum(sig_d, axis=1,
                                  keepdims=True).astype(jnp.float32)
        agg_ref[...] = agg

        stok = ((ssum + 0.5 * n_groups) * (1.0 / n_groups)) * m_ref[...]
        num = jnp.sum(stok)
        den = jnp.sum(m_ref[...])
        lane = jax.lax.broadcasted_iota(jnp.int32, (1, 1, 128), 2)
        vec = jnp.where(lane == 0, num, jnp.where(lane == 1, den, 0.0))
        prev = jnp.where(t % tpb == 0, jnp.zeros((1, 1, 128), jnp.float32),
                         sacc_ref[...])
        sacc_ref[...] = prev + vec


def _k2_body(hs_ref, agg_ref, w_ref, gam_ref, bet_ref, g_ref, out_ref):
    fc = jax.lax.dot_general(agg_ref[...], w_ref[...],
                             (((1,), (1,)), ((), ())),
                             preferred_element_type=jnp.float32)
    mod = hs_ref[...] + g_ref[...] * fc
    mu = jnp.mean(mod, axis=1, keepdims=True)
    d = mod - mu
    var = jnp.mean(d * d, axis=1, keepdims=True)
    out_ref[...] = d * jax.lax.rsqrt(var + 1e-5) * gam_ref[...] + bet_ref[...]


def kernel(hidden_states, attention_mask, feature_directions, group_embeddings,
           mixer_W, ln_gamma, ln_beta, safety_threshold):
    B, S, D = hidden_states.shape
    N = feature_directions.shape[0]
    G, F = group_embeddings.shape
    R = N // G
    topk = max(1, int(N * _SPARSITY))

    TB = 1024
    while (S % TB) != 0:
        TB //= 2
    NT = (B * S) // TB
    TPB = NT // B
    F_TILE = min(4096, G)
    while (G % F_TILE) != 0:
        F_TILE //= 2
    RPT = G // F_TILE
    F_TILES = R * RPT

    hs2 = hidden_states.reshape(B * S, D)
    mask2 = attention_mask.reshape(B * S, 1)
    hs_bf = (hs2 * mask2).astype(jnp.bfloat16)
    fd_perm = (feature_directions.reshape(G, R, D).transpose(1, 0, 2)
               .reshape(N, D).astype(jnp.bfloat16))

    k1 = pl.pallas_call(
        functools.partial(_k1_body, f_tiles=F_TILES, rpt=RPT, f_tile=F_TILE,
                          topk=topk, n_groups=G, tpb=TPB),
        grid=(NT, F_TILES),
        in_specs=[
            pl.BlockSpec((TB, D), lambda t, f: (t, 0)),
            pl.BlockSpec((F_TILE, D), lambda t, f: (f, 0)),
            pl.BlockSpec((TB, 1), lambda t, f: (t, 0)),
            pl.BlockSpec((G, F), lambda t, f: (0, 0)),
        ],
        out_specs=[
            pl.BlockSpec((TB, F), lambda t, f: (t, 0)),
            pl.BlockSpec((1, 1, 128), lambda t, f: (t // TPB, 0, 0)),
        ],
        out_shape=[
            jax.ShapeDtypeStruct((B * S, F), jnp.float32),
            jax.ShapeDtypeStruct((B, 1, 128), jnp.float32),
        ],
        scratch_shapes=[pltpu.VMEM((F_TILES, TB, F_TILE), jnp.bfloat16),
                        pltpu.VMEM((F_TILES, TB, 1), jnp.float32)],
        compiler_params=pltpu.CompilerParams(
            dimension_semantics=("arbitrary", "arbitrary")),
    )
    agg, sacc = k1(hs_bf, fd_perm, mask2,
                   group_embeddings.astype(jnp.bfloat16))

    num = sacc[:, 0, 0]
    den = jnp.maximum(sacc[:, 0, 1], 1.0)
    safety_scores = num / den
    gates = jax.nn.sigmoid(safety_scores - safety_threshold)
    g_tok = jnp.broadcast_to(gates[:, None], (B, S)).reshape(B * S, 1)

    TB2 = 512
    while ((B * S) % TB2) != 0:
        TB2 //= 2

    k2 = pl.pallas_call(
        _k2_body,
        grid=((B * S) // TB2,),
        in_specs=[
            pl.BlockSpec((TB2, D), lambda t: (t, 0)),
            pl.BlockSpec((TB2, F), lambda t: (t, 0)),
            pl.BlockSpec((D, F), lambda t: (0, 0)),
            pl.BlockSpec((1, D), lambda t: (0, 0)),
            pl.BlockSpec((1, D), lambda t: (0, 0)),
            pl.BlockSpec((TB2, 1), lambda t: (t, 0)),
        ],
        out_specs=pl.BlockSpec((TB2, D), lambda t: (t, 0)),
        out_shape=jax.ShapeDtypeStruct((B * S, D), jnp.float32),
        compiler_params=pltpu.CompilerParams(
            dimension_semantics=("arbitrary",)),
    )
    out2 = k2(hs2, agg, mixer_W, ln_gamma.reshape(1, D), ln_beta.reshape(1, D),
              g_tok)

    return out2.reshape(B, S, D), safety_scores


# speedup vs baseline: 1.4127x; 1.1582x over previous
import functools

import jax
import jax.numpy as jnp
from jax.experimental import pallas as pl
from jax.experimental.pallas import tpu as pltpu

_SPARSITY = 0.01
_BISECT_ITERS = 8


def _k1_body(hs_ref, fd_ref, m_ref, ge_ref, agg_ref, sacc_ref, acts_ref,
             max_ref, *, f_tiles, rpt, f_tile, topk, n_groups, tpb):
    t = pl.program_id(0)
    f = pl.program_id(1)

    a = jax.lax.dot_general(hs_ref[...], fd_ref[...],
                            (((1,), (1,)), ((), ())),
                            preferred_element_type=jnp.float32)
    a = jnp.maximum(a, 0.0)
    acts_ref[f] = a.astype(jnp.bfloat16)
    max_ref[f] = jnp.max(a, axis=1, keepdims=True)

    @pl.when(f == f_tiles - 1)
    def _finish():
        hi = max_ref[0]
        for j in range(1, f_tiles):
            hi = jnp.maximum(hi, max_ref[j])
        lo = jnp.zeros_like(hi)
        kf = jnp.float32(topk)

        def body(_, lh):
            lo, hi = lh
            mid16 = (0.5 * (lo + hi)).astype(jnp.bfloat16)
            msum = None
            for j in range(f_tiles):
                msk = jnp.where(acts_ref[j] >= mid16,
                                jnp.bfloat16(1), jnp.bfloat16(0))
                msum = msk if msum is None else msum + msk
            cnt = jnp.sum(msum, axis=1, keepdims=True).astype(jnp.float32)
            pred = cnt >= kf
            mid = mid16.astype(jnp.float32)
            return jnp.where(pred, mid, lo), jnp.where(pred, hi, mid)

        lo, hi = jax.lax.fori_loop(0, _BISECT_ITERS, body, (lo, hi))
        lo16 = lo.astype(jnp.bfloat16)

        agg = jnp.zeros(agg_ref.shape, jnp.float32)
        ssum = jnp.zeros((agg_ref.shape[0], 1), jnp.float32)
        for j in range(rpt):
            va = acts_ref[j]
            vb = acts_ref[rpt + j]
            vc = acts_ref[2 * rpt + j]
            va = jnp.where(va >= lo16, va, jnp.zeros_like(va))
            vb = jnp.where(vb >= lo16, vb, jnp.zeros_like(vb))
            vc = jnp.where(vc >= lo16, vc, jnp.zeros_like(vc))
            med = (jnp.maximum(jnp.minimum(va, vb),
                               jnp.minimum(jnp.maximum(va, vb), vc)))
            agg = agg + jax.lax.dot_general(
                med, ge_ref[j * f_tile:(j + 1) * f_tile, :],
                (((1,), (0,)), ((), ())),
                preferred_element_type=jnp.float32)
            sig_d = jax.nn.sigmoid(med) - jnp.bfloat16(0.5)
            ssum = ssum + jnp.sum(sig_d, axis=1,
                                  keepdims=True).astype(jnp.float32)
        agg_ref[...] = agg

        stok = ((ssum + 0.5 * n_groups) * (1.0 / n_groups)) * m_ref[...]
        num = jnp.sum(stok)
        den = jnp.sum(m_ref[...])
        lane = jax.lax.broadcasted_iota(jnp.int32, (1, 1, 128), 2)
        vec = jnp.where(lane == 0, num, jnp.where(lane == 1, den, 0.0))
        prev = jnp.where(t % tpb == 0, jnp.zeros((1, 1, 128), jnp.float32),
                         sacc_ref[...])
        sacc_ref[...] = prev + vec


def _k2_body(hs_ref, agg_ref, w_ref, gam_ref, bet_ref, g_ref, out_ref):
    fc = jax.lax.dot_general(agg_ref[...], w_ref[...],
                             (((1,), (1,)), ((), ())),
                             preferred_element_type=jnp.float32)
    mod = hs_ref[...] + g_ref[...] * fc
    mu = jnp.mean(mod, axis=1, keepdims=True)
    d = mod - mu
    var = jnp.mean(d * d, axis=1, keepdims=True)
    out_ref[...] = d * jax.lax.rsqrt(var + 1e-5) * gam_ref[...] + bet_ref[...]


def kernel(hidden_states, attention_mask, feature_directions, group_embeddings,
           mixer_W, ln_gamma, ln_beta, safety_threshold):
    B, S, D = hidden_states.shape
    N = feature_directions.shape[0]
    G, F = group_embeddings.shape
    R = N // G
    topk = max(1, int(N * _SPARSITY))

    TB = 1024
    while (S % TB) != 0:
        TB //= 2
    NT = (B * S) // TB
    TPB = NT // B
    F_TILE = min(4096, G)
    while (G % F_TILE) != 0:
        F_TILE //= 2
    RPT = G // F_TILE
    F_TILES = R * RPT

    hs2 = hidden_states.reshape(B * S, D)
    mask2 = attention_mask.reshape(B * S, 1)
    hs_bf = (hs2 * mask2).astype(jnp.bfloat16)
    fd_perm = (feature_directions.reshape(G, R, D).transpose(1, 0, 2)
               .reshape(N, D).astype(jnp.bfloat16))

    k1 = pl.pallas_call(
        functools.partial(_k1_body, f_tiles=F_TILES, rpt=RPT, f_tile=F_TILE,
                          topk=topk, n_groups=G, tpb=TPB),
        grid=(NT, F_TILES),
        in_specs=[
            pl.BlockSpec((TB, D), lambda t, f: (t, 0)),
            pl.BlockSpec((F_TILE, D), lambda t, f: (f, 0)),
            pl.BlockSpec((TB, 1), lambda t, f: (t, 0)),
            pl.BlockSpec((G, F), lambda t, f: (0, 0)),
        ],
        out_specs=[
            pl.BlockSpec((TB, F), lambda t, f: (t, 0)),
            pl.BlockSpec((1, 1, 128), lambda t, f: (t // TPB, 0, 0)),
        ],
        out_shape=[
            jax.ShapeDtypeStruct((B * S, F), jnp.float32),
            jax.ShapeDtypeStruct((B, 1, 128), jnp.float32),
        ],
        scratch_shapes=[pltpu.VMEM((F_TILES, TB, F_TILE), jnp.bfloat16),
                        pltpu.VMEM((F_TILES, TB, 1), jnp.float32)],
        compiler_params=pltpu.CompilerParams(
            dimension_semantics=("arbitrary", "arbitrary")),
    )
    agg, sacc = k1(hs_bf, fd_perm, mask2,
                   group_embeddings.astype(jnp.bfloat16))

    num = sacc[:, 0, 0]
    den = jnp.maximum(sacc[:, 0, 1], 1.0)
    safety_scores = num / den
    gates = jax.nn.sigmoid(safety_scores - safety_threshold)
    g_tok = jnp.broadcast_to(gates[:, None], (B, S)).reshape(B * S, 1)

    TB2 = 512
    while ((B * S) % TB2) != 0:
        TB2 //= 2

    k2 = pl.pallas_call(
        _k2_body,
        grid=((B * S) // TB2,),
        in_specs=[
            pl.BlockSpec((TB2, D), lambda t: (t, 0)),
            pl.BlockSpec((TB2, F), lambda t: (t, 0)),
            pl.BlockSpec((D, F), lambda t: (0, 0)),
            pl.BlockSpec((1, D), lambda t: (0, 0)),
            pl.BlockSpec((1, D), lambda t: (0, 0)),
            pl.BlockSpec((TB2, 1), lambda t: (t, 0)),
        ],
        out_specs=pl.BlockSpec((TB2, D), lambda t: (t, 0)),
        out_shape=jax.ShapeDtypeStruct((B * S, D), jnp.float32),
        compiler_params=pltpu.CompilerParams(
            dimension_semantics=("arbitrary",)),
    )
    out2 = k2(hs2, agg, mixer_W, ln_gamma.reshape(1, D), ln_beta.reshape(1, D),
              g_tok)

    return out2.reshape(B, S, D), safety_scores
